# baseline (speedup 1.0000x reference)
import jax
import jax.numpy as jnp
from jax.experimental import pallas as pl
from jax.experimental.pallas import tpu as pltpu

_TIME = 10000
_OUT = 64
_IN = 64


def _body(idx_ref, mat_ref, out_ref):
    out_ref[...] = mat_ref[0]


@jax.jit
def _lookup(matrix, idx):
    grid_spec = pltpu.PrefetchScalarGridSpec(
        num_scalar_prefetch=1,
        grid=(1,),
        in_specs=[
            pl.BlockSpec((1, _OUT, _IN), lambda i, idx_ref: (idx_ref[0], 0, 0)),
        ],
        out_specs=pl.BlockSpec((_OUT, _IN), lambda i, idx_ref: (0, 0)),
    )
    return pl.pallas_call(
        _body,
        grid_spec=grid_spec,
        out_shape=jax.ShapeDtypeStruct((_OUT, _IN), jnp.float32),
    )(idx, matrix)


def kernel(matrix, t):
    return _lookup(matrix, t.astype(jnp.int32).reshape(1))


# speedup vs baseline: 80.8133x; 80.8133x over previous
import jax
import jax.numpy as jnp
from jax.experimental import pallas as pl
from jax.experimental.pallas import tpu as pltpu

_TIME = 10000
_OUT = 64
_IN = 64
_LANES = 128


def _body(idx_ref, mat_ref, out_ref):
    j = idx_ref[0] % _LANES
    blk = mat_ref[...]
    lane = jax.lax.broadcasted_iota(jnp.int32, (_OUT, _IN, _LANES), 2)
    out_ref[...] = jnp.sum(jnp.where(lane == j, blk, 0.0), axis=2)


@jax.jit
def _lookup(mat_t, idx):
    grid_spec = pltpu.PrefetchScalarGridSpec(
        num_scalar_prefetch=1,
        grid=(1,),
        in_specs=[
            pl.BlockSpec(
                (_OUT, _IN, _LANES), lambda i, idx_ref: (0, 0, idx_ref[0] // _LANES)
            ),
        ],
        out_specs=pl.BlockSpec((_OUT, _IN), lambda i, idx_ref: (0, 0)),
    )
    return pl.pallas_call(
        _body,
        grid_spec=grid_spec,
        out_shape=jax.ShapeDtypeStruct((_OUT, _IN), jnp.float32),
    )(idx, mat_t)


def kernel(matrix, t):
    mat_t = jnp.transpose(matrix, (1, 2, 0))
    return _lookup(mat_t, t.astype(jnp.int32).reshape(1))
